# baseline (speedup 1.0000x reference)
import jax
import jax.numpy as jnp
from jax.experimental import pallas as pl
from jax.experimental.pallas import tpu as pltpu


def _pass1_body(adj_ref, x_ref, w1_ref, b_ref, w_ref,
                s_next_ref, adj16_ref, s1_scr):
    i = pl.program_id(0)

    @pl.when(i == 0)
    def _():
        s1_scr[...] = jnp.dot(
            x_ref[...].astype(jnp.bfloat16), w1_ref[...],
            preferred_element_type=jnp.float32).astype(jnp.bfloat16)

    a16 = adj_ref[...].astype(jnp.bfloat16)
    adj16_ref[...] = a16
    acc = jnp.dot(a16, s1_scr[...], preferred_element_type=jnp.float32)
    h = jnp.maximum(acc + b_ref[...], 0.0).astype(jnp.bfloat16)
    s_next_ref[...] = jnp.dot(
        h, w_ref[...], preferred_element_type=jnp.float32).astype(jnp.bfloat16)


def _pass23_body(adj16_ref, s2_ref, b2_ref, w3_ref, b3_ref,
                 out_ref, s3_scr):
    p = pl.program_id(0)
    i = pl.program_id(1)
    bi = adj16_ref.shape[0]

    @pl.when(p == 0)
    def _():
        acc = jnp.dot(adj16_ref[...], s2_ref[...],
                      preferred_element_type=jnp.float32)
        h = jnp.maximum(acc + b2_ref[...], 0.0).astype(jnp.bfloat16)
        s3_scr[pl.ds(i * bi, bi), :] = jnp.dot(
            h, w3_ref[...], preferred_element_type=jnp.float32
        ).astype(jnp.bfloat16)

    @pl.when(p == 1)
    def _():
        acc = jnp.dot(adj16_ref[...], s3_scr[...],
                      preferred_element_type=jnp.float32)
        out_ref[...] = acc + b3_ref[...]


def _rows_block(n, target):
    for bi in range(min(target, n), 7, -1):
        if n % bi == 0 and bi % 8 == 0:
            return bi
    return n


def kernel(x, adj, W1, b1, W2, b2, W3, b3):
    n, in_c = x.shape
    h1 = W1.shape[1]
    h2 = W2.shape[1]
    out_c = W3.shape[1]
    f32 = jnp.float32
    bf16 = jnp.bfloat16

    bi1 = _rows_block(n, 200)
    bi = _rows_block(n, 800)
    nblk1 = n // bi1
    nblk = n // bi

    w1_16 = W1.astype(bf16)
    w2_16 = W2.astype(bf16)
    w3_16 = W3.astype(bf16)
    b1r = b1.reshape(1, h1)
    b2r = b2.reshape(1, h2)
    b3r = b3.reshape(1, out_c)

    s2, adj16 = pl.pallas_call(
        _pass1_body,
        grid=(nblk1,),
        in_specs=[pl.BlockSpec((bi1, n), lambda i: (i, 0)),
                  pl.BlockSpec((n, in_c), lambda i: (0, 0)),
                  pl.BlockSpec((in_c, h1), lambda i: (0, 0)),
                  pl.BlockSpec((1, h1), lambda i: (0, 0)),
                  pl.BlockSpec((h1, h2), lambda i: (0, 0))],
        out_specs=[pl.BlockSpec((bi1, h2), lambda i: (i, 0)),
                   pl.BlockSpec((bi1, n), lambda i: (i, 0))],
        out_shape=[jax.ShapeDtypeStruct((n, h2), bf16),
                   jax.ShapeDtypeStruct((n, n), bf16)],
        scratch_shapes=[pltpu.VMEM((n, h1), bf16)],
    )(adj, x, w1_16, b1r, w2_16)

    out = pl.pallas_call(
        _pass23_body,
        grid=(2, nblk),
        in_specs=[pl.BlockSpec((bi, n), lambda p, i: (i, 0)),
                  pl.BlockSpec((n, h2), lambda p, i: (0, 0)),
                  pl.BlockSpec((1, h2), lambda p, i: (0, 0)),
                  pl.BlockSpec((h2, out_c), lambda p, i: (0, 0)),
                  pl.BlockSpec((1, out_c), lambda p, i: (0, 0))],
        out_specs=pl.BlockSpec((bi, out_c),
                               lambda p, i: (jnp.where(p == 1, i, 0), 0)),
        out_shape=jax.ShapeDtypeStruct((n, out_c), f32),
        scratch_shapes=[pltpu.VMEM((n, out_c), bf16)],
    )(adj16, s2, b2r, w3_16, b3r)

    return out


# speedup vs baseline: 1.0672x; 1.0672x over previous
import jax
import jax.numpy as jnp
from jax.experimental import pallas as pl
from jax.experimental.pallas import tpu as pltpu


def _pass1_body(adj_ref, x_ref, w1_ref, b_ref, w_ref,
                s_next_ref, adj16_ref, s1_scr):
    i = pl.program_id(0)

    @pl.when(i == 0)
    def _():
        s1_scr[...] = jnp.dot(
            x_ref[...].astype(jnp.bfloat16), w1_ref[...],
            preferred_element_type=jnp.float32).astype(jnp.bfloat16)

    a16 = adj_ref[...].astype(jnp.bfloat16)
    adj16_ref[...] = a16
    acc = jnp.dot(a16, s1_scr[...], preferred_element_type=jnp.float32)
    h = jnp.maximum(acc + b_ref[...], 0.0).astype(jnp.bfloat16)
    s_next_ref[...] = jnp.dot(
        h, w_ref[...], preferred_element_type=jnp.float32).astype(jnp.bfloat16)


def _pass23_body(adj16_ref, s2_ref, b2_ref, w3_ref, b3_ref,
                 out_ref, s3_scr):
    p = pl.program_id(0)
    i = pl.program_id(1)
    bi = adj16_ref.shape[0]

    @pl.when(p == 0)
    def _():
        acc = jnp.dot(adj16_ref[...], s2_ref[...],
                      preferred_element_type=jnp.float32)
        h = jnp.maximum(acc + b2_ref[...], 0.0).astype(jnp.bfloat16)
        s3_scr[pl.ds(i * bi, bi), :] = jnp.dot(
            h, w3_ref[...], preferred_element_type=jnp.float32
        ).astype(jnp.bfloat16)

    @pl.when(p == 1)
    def _():
        acc = jnp.dot(adj16_ref[...], s3_scr[...],
                      preferred_element_type=jnp.float32)
        out_ref[...] = acc + b3_ref[...]


def _rows_block(n, target):
    for bi in range(min(target, n), 7, -1):
        if n % bi == 0 and bi % 8 == 0:
            return bi
    return n


def kernel(x, adj, W1, b1, W2, b2, W3, b3):
    n, in_c = x.shape
    h1 = W1.shape[1]
    h2 = W2.shape[1]
    out_c = W3.shape[1]
    f32 = jnp.float32
    bf16 = jnp.bfloat16

    bi1 = _rows_block(n, 400)
    bi = _rows_block(n, 1000)
    nblk1 = n // bi1
    nblk = n // bi

    w1_16 = W1.astype(bf16)
    w2_16 = W2.astype(bf16)
    w3_16 = W3.astype(bf16)
    b1r = b1.reshape(1, h1)
    b2r = b2.reshape(1, h2)
    b3r = b3.reshape(1, out_c)

    s2, adj16 = pl.pallas_call(
        _pass1_body,
        grid=(nblk1,),
        in_specs=[pl.BlockSpec((bi1, n), lambda i: (i, 0)),
                  pl.BlockSpec((n, in_c), lambda i: (0, 0),
                               pipeline_mode=pl.Buffered(buffer_count=1)),
                  pl.BlockSpec((in_c, h1), lambda i: (0, 0)),
                  pl.BlockSpec((1, h1), lambda i: (0, 0)),
                  pl.BlockSpec((h1, h2), lambda i: (0, 0))],
        out_specs=[pl.BlockSpec((bi1, h2), lambda i: (i, 0)),
                   pl.BlockSpec((bi1, n), lambda i: (i, 0))],
        out_shape=[jax.ShapeDtypeStruct((n, h2), bf16),
                   jax.ShapeDtypeStruct((n, n), bf16)],
        scratch_shapes=[pltpu.VMEM((n, h1), bf16)],
    )(adj, x, w1_16, b1r, w2_16)

    out = pl.pallas_call(
        _pass23_body,
        grid=(2, nblk),
        in_specs=[pl.BlockSpec((bi, n), lambda p, i: (i, 0)),
                  pl.BlockSpec((n, h2), lambda p, i: (0, 0)),
                  pl.BlockSpec((1, h2), lambda p, i: (0, 0)),
                  pl.BlockSpec((h2, out_c), lambda p, i: (0, 0)),
                  pl.BlockSpec((1, out_c), lambda p, i: (0, 0))],
        out_specs=pl.BlockSpec((bi, out_c),
                               lambda p, i: (jnp.where(p == 1, i, 0), 0)),
        out_shape=jax.ShapeDtypeStruct((n, out_c), f32),
        scratch_shapes=[pltpu.VMEM((n, out_c), bf16)],
    )(adj16, s2, b2r, w3_16, b3r)

    return out
